# baseline (speedup 1.0000x reference)
import functools

import jax
import jax.numpy as jnp
from jax import lax
from jax.experimental import pallas as pl
from jax.experimental.pallas import tpu as pltpu
from jax.experimental.pallas import tpu_sc as plsc

DIM = 32
ROWS = 16384
COLS = 26
B = ROWS * COLS
NW = 32
BPW = B // NW
CH = 832
NCH = BPW // CH
NBUF = 4

_mesh = plsc.VectorSubcoreMesh(core_axis_name="c", subcore_axis_name="s")


@functools.partial(
    pl.kernel,
    mesh=_mesh,
    out_type=jax.ShapeDtypeStruct((B, DIM), jnp.float32),
    scratch_types=[
        pltpu.VMEM((BPW,), jnp.int32),
        pltpu.VMEM((NBUF, CH, DIM), jnp.float32),
        pltpu.SemaphoreType.DMA((NBUF,)),
        pltpu.SemaphoreType.DMA((NBUF,)),
    ],
    compiler_params=pltpu.CompilerParams(use_tc_tiling_on_sc=False),
)
def _emb_lookup(x_hbm, table_hbm, out_hbm, idx_v, rows_v, gsem, osem):
    wid = lax.axis_index("s") * 2 + lax.axis_index("c")
    base = wid * BPW

    pltpu.sync_copy(x_hbm.at[pl.ds(base, BPW)], idx_v)

    gh = [None] * NCH
    oh = [None] * NCH
    for i in range(NCH):
        b = i % NBUF
        if i >= NBUF:
            oh[i - NBUF].wait()
        gh[i] = pltpu.async_copy(
            table_hbm.at[idx_v.at[pl.ds(i * CH, CH)]], rows_v.at[b], gsem.at[b]
        )
        if i >= 1:
            bp = (i - 1) % NBUF
            gh[i - 1].wait()
            oh[i - 1] = pltpu.async_copy(
                rows_v.at[bp], out_hbm.at[pl.ds(base + (i - 1) * CH, CH)], osem.at[bp]
            )
    bl = (NCH - 1) % NBUF
    gh[NCH - 1].wait()
    oh[NCH - 1] = pltpu.async_copy(
        rows_v.at[bl], out_hbm.at[pl.ds(base + (NCH - 1) * CH, CH)], osem.at[bl]
    )
    for i in range(NCH - NBUF, NCH):
        oh[i].wait()


def kernel(x, table):
    xf = x.reshape(B).astype(jnp.int32)
    out = _emb_lookup(xf, table)
    return out.reshape(ROWS, COLS, DIM)


# speedup vs baseline: 1.0001x; 1.0001x over previous
import functools

import jax
import jax.numpy as jnp
from jax import lax
from jax.experimental import pallas as pl
from jax.experimental.pallas import tpu as pltpu
from jax.experimental.pallas import tpu_sc as plsc

DIM = 32
ROWS = 16384
COLS = 26
B = ROWS * COLS
NW = 32
BPW = B // NW
CH = 416
NCH = BPW // CH
NBUF = 8
NGIF = 4

_mesh = plsc.VectorSubcoreMesh(core_axis_name="c", subcore_axis_name="s")


@functools.partial(
    pl.kernel,
    mesh=_mesh,
    out_type=jax.ShapeDtypeStruct((B, DIM), jnp.float32),
    scratch_types=[
        pltpu.VMEM((BPW,), jnp.int32),
        pltpu.VMEM((NBUF, CH, DIM), jnp.float32),
        pltpu.SemaphoreType.DMA((NBUF,)),
        pltpu.SemaphoreType.DMA((NBUF,)),
    ],
    compiler_params=pltpu.CompilerParams(use_tc_tiling_on_sc=False),
)
def _emb_lookup(x_hbm, table_hbm, out_hbm, idx_v, rows_v, gsem, osem):
    wid = lax.axis_index("s") * 2 + lax.axis_index("c")
    base = wid * BPW

    pltpu.sync_copy(x_hbm.at[pl.ds(base, BPW)], idx_v)

    def fire_gather(j):
        return pltpu.async_copy(
            table_hbm.at[idx_v.at[pl.ds(j * CH, CH)]],
            rows_v.at[j % NBUF],
            gsem.at[j % NBUF],
        )

    gh = [None] * NCH
    oh = [None] * NCH
    for j in range(NGIF):
        gh[j] = fire_gather(j)
    for i in range(NCH):
        b = i % NBUF
        gh[i].wait()
        oh[i] = pltpu.async_copy(
            rows_v.at[b], out_hbm.at[pl.ds(base + i * CH, CH)], osem.at[b]
        )
        j = i + NGIF
        if j < NCH:
            if j >= NBUF:
                oh[j - NBUF].wait()
            gh[j] = fire_gather(j)
    for i in range(max(0, NCH - NBUF), NCH):
        oh[i].wait()


def kernel(x, table):
    xf = x.reshape(B).astype(jnp.int32)
    out = _emb_lookup(xf, table)
    return out.reshape(ROWS, COLS, DIM)
